# baseline (speedup 1.0000x reference)
import functools

import jax
import jax.numpy as jnp
from jax import lax
from jax.experimental import pallas as pl
from jax.experimental.pallas import tpu as pltpu
from jax.experimental.pallas import tpu_sc as plsc

D = 128
DH = 64
L = 16
NC = 2
NS = 16
NW = NC * NS
N_PAD = 10240
ROWS_PER_TILE = N_PAD // NS
DEG_ROWS = N_PAD // L
K_E = 128
K_B = 80
B_PAD = 10240
CB = B_PAD // (NW * K_B)
MB = 1000

_mesh = plsc.VectorSubcoreMesh(core_axis_name="c", subcore_axis_name="s")
_sc_params = pltpu.CompilerParams(use_tc_tiling_on_sc=False,
                                  needs_layout_passes=False)


def _edge_kernel(src_hbm, dst_hbm, featS, zsum_hbm, zdeg_hbm, iota_hbm,
                 sum_out, deg_out,
                 src_v, dst_v, rows0, rows1, rows2, rows3,
                 degl_v, degf_v, iota_v,
                 acc_sp, deg_sp, g0, g1, g2, g3, s0, s1, s2, s3):
    c = lax.axis_index("c")
    s = lax.axis_index("s")
    rows = pl.ds(s * ROWS_PER_TILE, ROWS_PER_TILE)
    rows_out = pl.ds(c * N_PAD + s * ROWS_PER_TILE, ROWS_PER_TILE)
    drows = pl.ds(s * (DEG_ROWS // NS), DEG_ROWS // NS)
    pltpu.sync_copy(zsum_hbm.at[rows], acc_sp.at[rows])
    pltpu.sync_copy(zdeg_hbm.at[drows], deg_sp.at[drows])
    pltpu.sync_copy(zdeg_hbm, degl_v)
    pltpu.sync_copy(src_hbm.at[c, s], src_v)
    pltpu.sync_copy(dst_hbm.at[s], dst_v)
    pltpu.sync_copy(iota_hbm, iota_v)
    plsc.subcore_barrier()

    n_chunks = src_v.shape[0]
    bufs = [rows0, rows1, rows2, rows3]
    gsem = [g0, g1, g2, g3]
    ssem = [s0, s1, s2, s3]
    ones16 = jnp.full((L,), 1.0, jnp.float32)

    def deg_chunk(j):
        def dseg(k, carry):
            dvec = dst_v[j, pl.ds(k * L, L)]
            plsc.addupdate_scatter(
                degl_v, [lax.shift_right_logical(dvec, 4), dvec & 15], ones16)
            return carry

        lax.fori_loop(0, K_E // L, dseg, 0)

    def start_g(j, b):
        pltpu.async_copy(featS.at[src_v.at[j]], bufs[b], gsem[b])

    def wait_g(b):
        pltpu.make_async_copy(featS.at[src_v.at[0]], bufs[b], gsem[b]).wait()

    def start_s(j, b):
        pltpu.async_copy(bufs[b], acc_sp.at[dst_v.at[j]], ssem[b], add=True)

    def wait_s(b):
        pltpu.make_async_copy(bufs[b], acc_sp.at[dst_v.at[0]], ssem[b]).wait()

    start_g(0, 0)
    start_g(1, 1)
    start_g(2, 2)
    wait_g(0)
    start_s(0, 0)
    deg_chunk(0)
    start_g(3, 3)
    wait_g(1)
    start_s(1, 1)
    deg_chunk(1)
    wait_s(0)
    start_g(4, 0)
    wait_g(2)
    start_s(2, 2)
    deg_chunk(2)
    wait_s(1)
    start_g(5, 1)
    wait_g(3)
    start_s(3, 3)
    deg_chunk(3)

    def quad(q, carry):
        jb = 4 * q
        for b in range(4):
            j = jb + b
            wait_s((b + 2) % 4)
            start_g(jnp.minimum(j + 2, n_chunks - 1), (b + 2) % 4)
            wait_g(b)
            start_s(j, b)
            deg_chunk(j)
        return carry

    lax.fori_loop(1, n_chunks // 4, quad, 0)
    wait_s(2)
    wait_s(3)
    wait_g(0)
    wait_g(1)

    def dmerge(r, carry):
        pltpu.sync_copy(degl_v.at[pl.ds(r * 128, 128)],
                        deg_sp.at[iota_v.at[r]], add=True)
        return carry

    lax.fori_loop(0, DEG_ROWS // 128, dmerge, 0)
    plsc.subcore_barrier()

    pltpu.sync_copy(acc_sp.at[rows], sum_out.at[rows_out])
    pltpu.sync_copy(deg_sp.at[drows], degf_v)

    def dexp(n, carry):
        val = plsc.load_gather(
            degf_v, [jnp.full((L,), n // L, jnp.int32),
                     jnp.full((L,), n % L, jnp.int32)])
        degl_v[n, :] = val
        return carry

    lax.fori_loop(0, ROWS_PER_TILE, dexp, 0)
    pltpu.sync_copy(degl_v.at[pl.ds(0, ROWS_PER_TILE)],
                    deg_out.at[pl.ds(s * ROWS_PER_TILE, ROWS_PER_TILE)])


def _combine_kernel(nodes_hbm, feat_hbm, sum_hbm, dg_hbm,
                    self_out, neigh_out,
                    nodes_v, idx2_v, sf_v, pa_v, pb_v, d_v, nout_v, sem):
    c = lax.axis_index("c")
    s = lax.axis_index("s")
    wid = s * NC + c
    pltpu.sync_copy(nodes_hbm.at[wid], nodes_v)
    base = wid * (CB * K_B)

    def chunk(j, carry):
        idx = nodes_v.at[j]

        def shift(kk, carry2):
            sl = pl.ds(kk * L, L)
            idx2_v[sl] = nodes_v[j, sl] + N_PAD
            return carry2

        lax.fori_loop(0, K_B // L, shift, 0)

        cps = [
            pltpu.async_copy(feat_hbm.at[idx], sf_v, sem),
            pltpu.async_copy(sum_hbm.at[idx], pa_v, sem),
            pltpu.async_copy(sum_hbm.at[idx2_v], pb_v, sem),
            pltpu.async_copy(dg_hbm.at[idx], d_v, sem),
        ]
        for cp in cps:
            cp.wait()

        def row(i, carry2):
            inv = 1.0 / jnp.maximum(d_v[i, :], 1.0)

            def seg(k, carry3):
                sl = pl.ds(k * L, L)
                nout_v[i, sl] = pa_v[i, sl] * inv
                nout_v[i, pl.ds(DH + k * L, L)] = pb_v[i, sl] * inv
                return carry3

            lax.fori_loop(0, DH // L, seg, 0)
            return carry2

        lax.fori_loop(0, K_B, row, 0)
        dst = pl.ds(base + j * K_B, K_B)
        pltpu.sync_copy(sf_v, self_out.at[dst])
        pltpu.sync_copy(nout_v, neigh_out.at[dst])
        return carry

    lax.fori_loop(0, CB, chunk, 0)


def _mlp_kernel(s_ref, n_ref, w_ref, b_ref, o_ref):
    xs = s_ref[...]
    xn = n_ref[...]
    w = w_ref[...]
    acc = lax.dot_general(xs, w[:, :D], (((1,), (1,)), ((), ())),
                          preferred_element_type=jnp.float32)
    acc += lax.dot_general(xn, w[:, D:], (((1,), (1,)), ((), ())),
                           preferred_element_type=jnp.float32)
    o_ref[...] = jnp.maximum(acc + b_ref[...], 0.0)


def _build_edge_kernel(ch2):
    f32 = jnp.float32
    i32 = jnp.int32
    return pl.kernel(
        _edge_kernel,
        out_type=[
            jax.ShapeDtypeStruct((2 * N_PAD, DH), f32),
            jax.ShapeDtypeStruct((N_PAD, 16), f32),
        ],
        mesh=_mesh,
        scratch_types=[
            pltpu.VMEM((ch2, K_E), i32),
            pltpu.VMEM((ch2, K_E), i32),
            pltpu.VMEM((K_E, DH), f32),
            pltpu.VMEM((K_E, DH), f32),
            pltpu.VMEM((K_E, DH), f32),
            pltpu.VMEM((K_E, DH), f32),
            pltpu.VMEM((DEG_ROWS, L), f32),
            pltpu.VMEM((DEG_ROWS // NS, L), f32),
            pltpu.VMEM((DEG_ROWS // 128, 128), i32),
            pltpu.VMEM_SHARED((N_PAD, DH), f32),
            pltpu.VMEM_SHARED((DEG_ROWS, L), f32),
            pltpu.SemaphoreType.DMA,
            pltpu.SemaphoreType.DMA,
            pltpu.SemaphoreType.DMA,
            pltpu.SemaphoreType.DMA,
            pltpu.SemaphoreType.DMA,
            pltpu.SemaphoreType.DMA,
            pltpu.SemaphoreType.DMA,
            pltpu.SemaphoreType.DMA,
        ],
        compiler_params=_sc_params,
    )


def _build_combine_kernel():
    f32 = jnp.float32
    return pl.kernel(
        _combine_kernel,
        out_type=[
            jax.ShapeDtypeStruct((B_PAD, D), f32),
            jax.ShapeDtypeStruct((B_PAD, D), f32),
        ],
        mesh=_mesh,
        scratch_types=[
            pltpu.VMEM((CB, K_B), jnp.int32),
            pltpu.VMEM((K_B,), jnp.int32),
            pltpu.VMEM((K_B, D), f32),
            pltpu.VMEM((K_B, DH), f32),
            pltpu.VMEM((K_B, DH), f32),
            pltpu.VMEM((K_B, 16), f32),
            pltpu.VMEM((K_B, D), f32),
            pltpu.SemaphoreType.DMA,
        ],
        compiler_params=_sc_params,
    )


def kernel(nodes, edge_index, features, W1, b1):
    E = edge_index.shape[1]
    B = nodes.shape[0]
    N = features.shape[0]
    f32 = jnp.float32
    i32 = jnp.int32

    e_per_tile = E // NS
    ch2 = -(-e_per_tile // (4 * K_E)) * 4
    e_pad = ch2 * K_E - e_per_tile

    src_t = jnp.pad(edge_index[0].astype(i32).reshape(NS, e_per_tile),
                    ((0, 0), (0, e_pad))).reshape(NS, ch2, K_E)
    dst_t = jnp.pad(edge_index[1].astype(i32).reshape(NS, e_per_tile),
                    ((0, 0), (0, e_pad)),
                    constant_values=N_PAD - 1).reshape(NS, ch2, K_E)
    src_both = jnp.stack([2 * src_t, 2 * src_t + 1])
    featS = features.reshape(2 * N, DH)
    nodes_r = jnp.pad(nodes.astype(i32), (0, B_PAD - B)).reshape(NW, CB, K_B)
    zsum = jnp.zeros((N_PAD, DH), f32)
    zdeg = jnp.zeros((DEG_ROWS, L), f32)
    iota = jnp.arange(DEG_ROWS, dtype=i32).reshape(DEG_ROWS // 128, 128)

    sum_out, deg_out = _build_edge_kernel(ch2)(
        src_both, dst_t, featS, zsum, zdeg, iota)

    self_f, neigh_f = _build_combine_kernel()(
        nodes_r, features, sum_out, deg_out)

    out = pl.pallas_call(
        _mlp_kernel,
        grid=(B // MB,),
        in_specs=[
            pl.BlockSpec((MB, D), lambda i: (i, 0)),
            pl.BlockSpec((MB, D), lambda i: (i, 0)),
            pl.BlockSpec((D, 2 * D), lambda i: (0, 0)),
            pl.BlockSpec((1, D), lambda i: (0, 0)),
        ],
        out_specs=pl.BlockSpec((MB, D), lambda i: (i, 0)),
        out_shape=jax.ShapeDtypeStruct((B, D), f32),
    )(self_f, neigh_f, W1, b1.reshape(1, D))

    return out


# speedup vs baseline: 1.0861x; 1.0861x over previous
import functools

import jax
import jax.numpy as jnp
from jax import lax
from jax.experimental import pallas as pl
from jax.experimental.pallas import tpu as pltpu
from jax.experimental.pallas import tpu_sc as plsc

D = 128
DH = 64
L = 16
NC = 2
NS = 16
NW = NC * NS
N_PAD = 10240
ROWS_PER_TILE = N_PAD // NS
DEG_ROWS = N_PAD // L
K_E = 128
K_B = 80
B_PAD = 10240
CB = B_PAD // (NW * K_B)
MB = 1000

_mesh = plsc.VectorSubcoreMesh(core_axis_name="c", subcore_axis_name="s")
_sc_params = pltpu.CompilerParams(use_tc_tiling_on_sc=False,
                                  needs_layout_passes=False)


def _edge_kernel(src_hbm, dst_hbm, featS, zsum_hbm, zdeg_hbm, iota_hbm,
                 sum_out, deg_out,
                 src_v, dst_v, rows0, rows1,
                 degl_v, degf_v, iota_v,
                 acc_sp, deg_sp, g0, g1):
    c = lax.axis_index("c")
    s = lax.axis_index("s")
    rows = pl.ds(s * ROWS_PER_TILE, ROWS_PER_TILE)
    rows_out = pl.ds(c * N_PAD + s * ROWS_PER_TILE, ROWS_PER_TILE)
    drows = pl.ds(s * (DEG_ROWS // NS), DEG_ROWS // NS)
    pltpu.sync_copy(zsum_hbm.at[rows], acc_sp.at[rows])
    pltpu.sync_copy(zdeg_hbm.at[drows], deg_sp.at[drows])
    pltpu.sync_copy(zdeg_hbm, degl_v)
    pltpu.sync_copy(src_hbm.at[c, s], src_v)
    pltpu.sync_copy(dst_hbm.at[s], dst_v)
    pltpu.sync_copy(iota_hbm, iota_v)
    plsc.subcore_barrier()

    n_chunks = src_v.shape[0]
    bufs = [rows0, rows1]
    gsem = [g0, g1]
    ones16 = jnp.full((L,), 1.0, jnp.float32)

    def deg_chunk(j):
        def dseg(k, carry):
            dvec = dst_v[j, pl.ds(k * L, L)]
            plsc.addupdate_scatter(
                degl_v, [lax.shift_right_logical(dvec, 4), dvec & 15], ones16)
            return carry

        lax.fori_loop(0, K_E // L, dseg, 0)

    def start_g(j, b):
        pltpu.async_copy(featS.at[src_v.at[j]], bufs[b], gsem[b])

    def wait_g(b):
        pltpu.make_async_copy(featS.at[src_v.at[0]], bufs[b], gsem[b]).wait()

    start_g(0, 0)

    def pair(j2, carry):
        j0 = 2 * j2
        j1 = j0 + 1
        start_g(j1, 1)
        wait_g(0)
        pltpu.sync_copy(rows0, acc_sp.at[dst_v.at[j0]], add=True)
        deg_chunk(j0)
        start_g(jnp.minimum(j0 + 2, n_chunks - 1), 0)
        wait_g(1)
        pltpu.sync_copy(rows1, acc_sp.at[dst_v.at[j1]], add=True)
        deg_chunk(j1)
        return carry

    lax.fori_loop(0, n_chunks // 2, pair, 0)
    wait_g(0)

    def dmerge(r, carry):
        pltpu.sync_copy(degl_v.at[pl.ds(r * 128, 128)],
                        deg_sp.at[iota_v.at[r]], add=True)
        return carry

    lax.fori_loop(0, DEG_ROWS // 128, dmerge, 0)
    plsc.subcore_barrier()

    pltpu.sync_copy(acc_sp.at[rows], sum_out.at[rows_out])
    pltpu.sync_copy(deg_sp.at[drows], degf_v)

    def dexp(n, carry):
        val = plsc.load_gather(
            degf_v, [jnp.full((L,), n // L, jnp.int32),
                     jnp.full((L,), n % L, jnp.int32)])
        degl_v[n, :] = val
        return carry

    lax.fori_loop(0, ROWS_PER_TILE, dexp, 0)
    pltpu.sync_copy(degl_v.at[pl.ds(0, ROWS_PER_TILE)],
                    deg_out.at[pl.ds(s * ROWS_PER_TILE, ROWS_PER_TILE)])


def _combine_kernel(nodes_hbm, feat_hbm, sum_hbm, dg_hbm,
                    self_out, neigh_out,
                    nodes_v, idx2_v, sf_v, pa_v, pb_v, d_v, nout_v, sem):
    c = lax.axis_index("c")
    s = lax.axis_index("s")
    wid = s * NC + c
    pltpu.sync_copy(nodes_hbm.at[wid], nodes_v)
    base = wid * (CB * K_B)

    def chunk(j, carry):
        idx = nodes_v.at[j]

        def shift(kk, carry2):
            sl = pl.ds(kk * L, L)
            idx2_v[sl] = nodes_v[j, sl] + N_PAD
            return carry2

        lax.fori_loop(0, K_B // L, shift, 0)

        cps = [
            pltpu.async_copy(feat_hbm.at[idx], sf_v, sem),
            pltpu.async_copy(sum_hbm.at[idx], pa_v, sem),
            pltpu.async_copy(sum_hbm.at[idx2_v], pb_v, sem),
            pltpu.async_copy(dg_hbm.at[idx], d_v, sem),
        ]
        for cp in cps:
            cp.wait()

        def row(i, carry2):
            inv = 1.0 / jnp.maximum(d_v[i, :], 1.0)

            def seg(k, carry3):
                sl = pl.ds(k * L, L)
                nout_v[i, sl] = pa_v[i, sl] * inv
                nout_v[i, pl.ds(DH + k * L, L)] = pb_v[i, sl] * inv
                return carry3

            lax.fori_loop(0, DH // L, seg, 0)
            return carry2

        lax.fori_loop(0, K_B, row, 0)
        dst = pl.ds(base + j * K_B, K_B)
        pltpu.sync_copy(sf_v, self_out.at[dst])
        pltpu.sync_copy(nout_v, neigh_out.at[dst])
        return carry

    lax.fori_loop(0, CB, chunk, 0)


def _mlp_kernel(s_ref, n_ref, w_ref, b_ref, o_ref):
    xs = s_ref[...]
    xn = n_ref[...]
    w = w_ref[...]
    acc = lax.dot_general(xs, w[:, :D], (((1,), (1,)), ((), ())),
                          preferred_element_type=jnp.float32)
    acc += lax.dot_general(xn, w[:, D:], (((1,), (1,)), ((), ())),
                           preferred_element_type=jnp.float32)
    o_ref[...] = jnp.maximum(acc + b_ref[...], 0.0)


def _build_edge_kernel(ch2):
    f32 = jnp.float32
    i32 = jnp.int32
    return pl.kernel(
        _edge_kernel,
        out_type=[
            jax.ShapeDtypeStruct((2 * N_PAD, DH), f32),
            jax.ShapeDtypeStruct((N_PAD, 16), f32),
        ],
        mesh=_mesh,
        scratch_types=[
            pltpu.VMEM((ch2, K_E), i32),
            pltpu.VMEM((ch2, K_E), i32),
            pltpu.VMEM((K_E, DH), f32),
            pltpu.VMEM((K_E, DH), f32),
            pltpu.VMEM((DEG_ROWS, L), f32),
            pltpu.VMEM((DEG_ROWS // NS, L), f32),
            pltpu.VMEM((DEG_ROWS // 128, 128), i32),
            pltpu.VMEM_SHARED((N_PAD, DH), f32),
            pltpu.VMEM_SHARED((DEG_ROWS, L), f32),
            pltpu.SemaphoreType.DMA,
            pltpu.SemaphoreType.DMA,
        ],
        compiler_params=_sc_params,
    )


def _build_combine_kernel():
    f32 = jnp.float32
    return pl.kernel(
        _combine_kernel,
        out_type=[
            jax.ShapeDtypeStruct((B_PAD, D), f32),
            jax.ShapeDtypeStruct((B_PAD, D), f32),
        ],
        mesh=_mesh,
        scratch_types=[
            pltpu.VMEM((CB, K_B), jnp.int32),
            pltpu.VMEM((K_B,), jnp.int32),
            pltpu.VMEM((K_B, D), f32),
            pltpu.VMEM((K_B, DH), f32),
            pltpu.VMEM((K_B, DH), f32),
            pltpu.VMEM((K_B, 16), f32),
            pltpu.VMEM((K_B, D), f32),
            pltpu.SemaphoreType.DMA,
        ],
        compiler_params=_sc_params,
    )


def kernel(nodes, edge_index, features, W1, b1):
    E = edge_index.shape[1]
    B = nodes.shape[0]
    N = features.shape[0]
    f32 = jnp.float32
    i32 = jnp.int32

    e_per_tile = E // NS
    ch2 = -(-e_per_tile // (4 * K_E)) * 4
    e_pad = ch2 * K_E - e_per_tile

    src_t = jnp.pad(edge_index[0].astype(i32).reshape(NS, e_per_tile),
                    ((0, 0), (0, e_pad))).reshape(NS, ch2, K_E)
    dst_t = jnp.pad(edge_index[1].astype(i32).reshape(NS, e_per_tile),
                    ((0, 0), (0, e_pad)),
                    constant_values=N_PAD - 1).reshape(NS, ch2, K_E)
    src_both = jnp.stack([2 * src_t, 2 * src_t + 1])
    featS = features.reshape(2 * N, DH)
    nodes_r = jnp.pad(nodes.astype(i32), (0, B_PAD - B)).reshape(NW, CB, K_B)
    zsum = jnp.zeros((N_PAD, DH), f32)
    zdeg = jnp.zeros((DEG_ROWS, L), f32)
    iota = jnp.arange(DEG_ROWS, dtype=i32).reshape(DEG_ROWS // 128, 128)

    sum_out, deg_out = _build_edge_kernel(ch2)(
        src_both, dst_t, featS, zsum, zdeg, iota)

    self_f, neigh_f = _build_combine_kernel()(
        nodes_r, features, sum_out, deg_out)

    out = pl.pallas_call(
        _mlp_kernel,
        grid=(B // MB,),
        in_specs=[
            pl.BlockSpec((MB, D), lambda i: (i, 0)),
            pl.BlockSpec((MB, D), lambda i: (i, 0)),
            pl.BlockSpec((D, 2 * D), lambda i: (0, 0)),
            pl.BlockSpec((1, D), lambda i: (0, 0)),
        ],
        out_specs=pl.BlockSpec((MB, D), lambda i: (i, 0)),
        out_shape=jax.ShapeDtypeStruct((B, D), f32),
    )(self_f, neigh_f, W1, b1.reshape(1, D))

    return out


# speedup vs baseline: 1.4867x; 1.3688x over previous
import functools

import jax
import jax.numpy as jnp
from jax import lax
from jax.experimental import pallas as pl
from jax.experimental.pallas import tpu as pltpu
from jax.experimental.pallas import tpu_sc as plsc

D = 128
DH = 64
L = 16
NC = 2
NS = 16
NW = NC * NS
N_PAD = 10240
ROWS_PER_TILE = N_PAD // NS
DEG_ROWS = N_PAD // L
K_E = 128
K_B = 80
B_PAD = 10240
CB = B_PAD // (NW * K_B)
MB = 1000

_mesh = plsc.VectorSubcoreMesh(core_axis_name="c", subcore_axis_name="s")
_sc_params = pltpu.CompilerParams(use_tc_tiling_on_sc=False,
                                  needs_layout_passes=False)


def _edge_kernel(src_hbm, dst_hbm, featS, zsum_hbm, zdeg_hbm, iota_hbm,
                 sum_out, deg_out,
                 src_v, dst_v, rows0, rows1,
                 degl_v, degf_v, iota_v,
                 acc_sp, deg_sp, g0, g1):
    c = lax.axis_index("c")
    s = lax.axis_index("s")
    rows = pl.ds(s * ROWS_PER_TILE, ROWS_PER_TILE)
    rows_out = pl.ds(c * N_PAD + s * ROWS_PER_TILE, ROWS_PER_TILE)
    drows = pl.ds(s * (DEG_ROWS // NS), DEG_ROWS // NS)
    pltpu.sync_copy(zsum_hbm.at[rows], acc_sp.at[rows])
    pltpu.sync_copy(zdeg_hbm.at[drows], deg_sp.at[drows])
    pltpu.sync_copy(zdeg_hbm, degl_v)
    pltpu.sync_copy(src_hbm.at[c, s], src_v)
    pltpu.sync_copy(dst_hbm.at[s], dst_v)
    pltpu.sync_copy(iota_hbm, iota_v)
    plsc.subcore_barrier()

    n_chunks = src_v.shape[0]
    bufs = [rows0, rows1]
    gsem = [g0, g1]
    ones16 = jnp.full((L,), 1.0, jnp.float32)

    def deg_chunk(j):
        def dseg(k, carry):
            dvec = dst_v[j, pl.ds(k * L, L)]
            plsc.addupdate_scatter(
                degl_v, [lax.shift_right_logical(dvec, 4), dvec & 15], ones16)
            return carry

        lax.fori_loop(0, K_E // L, dseg, 0)

    def start_g(j, b):
        pltpu.async_copy(featS.at[src_v.at[j]], bufs[b], gsem[b])

    def wait_g(b):
        pltpu.make_async_copy(featS.at[src_v.at[0]], bufs[b], gsem[b]).wait()

    start_g(0, 0)

    def pair(j2, carry):
        j0 = 2 * j2
        j1 = j0 + 1
        start_g(j1, 1)
        wait_g(0)
        pltpu.sync_copy(rows0, acc_sp.at[dst_v.at[j0]], add=True)
        deg_chunk(j0)
        start_g(jnp.minimum(j0 + 2, n_chunks - 1), 0)
        wait_g(1)
        pltpu.sync_copy(rows1, acc_sp.at[dst_v.at[j1]], add=True)
        deg_chunk(j1)
        return carry

    lax.fori_loop(0, n_chunks // 2, pair, 0)
    wait_g(0)

    def dmerge(r, carry):
        pltpu.sync_copy(degl_v.at[pl.ds(r * 128, 128)],
                        deg_sp.at[iota_v.at[r]], add=True)
        return carry

    lax.fori_loop(0, DEG_ROWS // 128, dmerge, 0)
    plsc.subcore_barrier()

    pltpu.sync_copy(acc_sp.at[rows], sum_out.at[rows_out])
    pltpu.sync_copy(deg_sp.at[drows], degf_v)

    def dexp(n, carry):
        val = plsc.load_gather(
            degf_v, [jnp.full((L,), n // L, jnp.int32),
                     jnp.full((L,), n % L, jnp.int32)])
        degl_v[n, :] = val
        return carry

    lax.fori_loop(0, ROWS_PER_TILE, dexp, 0)
    pltpu.sync_copy(degl_v.at[pl.ds(0, ROWS_PER_TILE)],
                    deg_out.at[pl.ds(s * ROWS_PER_TILE, ROWS_PER_TILE)])


def _combine_kernel(nodes_hbm, feat_hbm, sum_hbm, dg_hbm,
                    self_out, neigh_out,
                    nodes_v, idx2_v, sf_v, pa_v, pb_v, d_v, nout_v, sem):
    c = lax.axis_index("c")
    s = lax.axis_index("s")
    wid = s * NC + c
    pltpu.sync_copy(nodes_hbm.at[wid], nodes_v)
    base = wid * (CB * K_B)

    def chunk(j, carry):
        idx = nodes_v.at[j]

        def shift(kk, carry2):
            sl = pl.ds(kk * L, L)
            idx2_v[sl] = nodes_v[j, sl] + N_PAD
            return carry2

        lax.fori_loop(0, K_B // L, shift, 0)

        cps = [
            pltpu.async_copy(feat_hbm.at[idx], sf_v, sem),
            pltpu.async_copy(sum_hbm.at[idx], pa_v, sem),
            pltpu.async_copy(sum_hbm.at[idx2_v], pb_v, sem),
            pltpu.async_copy(dg_hbm.at[idx], d_v, sem),
        ]
        for cp in cps:
            cp.wait()

        def row(i, carry2):
            inv = 1.0 / jnp.maximum(d_v[i, :], 1.0)

            def seg(k, carry3):
                sl = pl.ds(k * L, L)
                nout_v[i, sl] = pa_v[i, sl] * inv
                nout_v[i, pl.ds(DH + k * L, L)] = pb_v[i, sl] * inv
                return carry3

            lax.fori_loop(0, DH // L, seg, 0)
            return carry2

        lax.fori_loop(0, K_B, row, 0)
        dst = pl.ds(base + j * K_B, K_B)
        pltpu.sync_copy(sf_v, self_out.at[dst])
        pltpu.sync_copy(nout_v, neigh_out.at[dst])
        return carry

    lax.fori_loop(0, CB, chunk, 0)


def _mlp_kernel(s_ref, n_ref, w_ref, b_ref, o_ref):
    xs = s_ref[...]
    xn = n_ref[...]
    w = w_ref[...]
    acc = lax.dot_general(xs, w[:, :D], (((1,), (1,)), ((), ())),
                          preferred_element_type=jnp.float32)
    acc += lax.dot_general(xn, w[:, D:], (((1,), (1,)), ((), ())),
                           preferred_element_type=jnp.float32)
    o_ref[...] = jnp.maximum(acc + b_ref[...], 0.0)


def _build_edge_kernel(ch2):
    f32 = jnp.float32
    i32 = jnp.int32
    return pl.kernel(
        _edge_kernel,
        out_type=[
            jax.ShapeDtypeStruct((2 * N_PAD, DH), f32),
            jax.ShapeDtypeStruct((N_PAD, 16), f32),
        ],
        mesh=_mesh,
        scratch_types=[
            pltpu.VMEM((ch2, K_E), i32),
            pltpu.VMEM((ch2, K_E), i32),
            pltpu.VMEM((K_E, DH), f32),
            pltpu.VMEM((K_E, DH), f32),
            pltpu.VMEM((DEG_ROWS, L), f32),
            pltpu.VMEM((DEG_ROWS // NS, L), f32),
            pltpu.VMEM((DEG_ROWS // 128, 128), i32),
            pltpu.VMEM_SHARED((N_PAD, DH), f32),
            pltpu.VMEM_SHARED((DEG_ROWS, L), f32),
            pltpu.SemaphoreType.DMA,
            pltpu.SemaphoreType.DMA,
        ],
        compiler_params=_sc_params,
    )


def _build_combine_kernel():
    f32 = jnp.float32
    return pl.kernel(
        _combine_kernel,
        out_type=[
            jax.ShapeDtypeStruct((B_PAD, D), f32),
            jax.ShapeDtypeStruct((B_PAD, D), f32),
        ],
        mesh=_mesh,
        scratch_types=[
            pltpu.VMEM((CB, K_B), jnp.int32),
            pltpu.VMEM((K_B,), jnp.int32),
            pltpu.VMEM((K_B, D), f32),
            pltpu.VMEM((K_B, DH), f32),
            pltpu.VMEM((K_B, DH), f32),
            pltpu.VMEM((K_B, 16), f32),
            pltpu.VMEM((K_B, D), f32),
            pltpu.SemaphoreType.DMA,
        ],
        compiler_params=_sc_params,
    )


def kernel(nodes, edge_index, features, W1, b1):
    E = edge_index.shape[1]
    B = nodes.shape[0]
    N = features.shape[0]
    f32 = jnp.float32
    i32 = jnp.int32

    e_per_tile = E // NS
    ch2 = -(-e_per_tile // (4 * K_E)) * 4
    e_pad = ch2 * K_E - e_per_tile

    src_t = jnp.pad(edge_index[0].astype(i32).reshape(NS, e_per_tile),
                    ((0, 0), (0, e_pad))).reshape(NS, ch2, K_E)
    dst_t = jnp.pad(edge_index[1].astype(i32).reshape(NS, e_per_tile),
                    ((0, 0), (0, e_pad)),
                    constant_values=N_PAD - 1).reshape(NS, ch2, K_E)
    src_both = jnp.stack([src_t, src_t + N])
    featS = jnp.concatenate([features[:, :DH], features[:, DH:]], axis=0)
    nodes_r = jnp.pad(nodes.astype(i32), (0, B_PAD - B)).reshape(NW, CB, K_B)
    zsum = jnp.zeros((N_PAD, DH), f32)
    zdeg = jnp.zeros((DEG_ROWS, L), f32)
    iota = jnp.arange(DEG_ROWS, dtype=i32).reshape(DEG_ROWS // 128, 128)

    sum_out, deg_out = _build_edge_kernel(ch2)(
        src_both, dst_t, featS, zsum, zdeg, iota)

    self_f, neigh_f = _build_combine_kernel()(
        nodes_r, features, sum_out, deg_out)

    out = pl.pallas_call(
        _mlp_kernel,
        grid=(B // MB,),
        in_specs=[
            pl.BlockSpec((MB, D), lambda i: (i, 0)),
            pl.BlockSpec((MB, D), lambda i: (i, 0)),
            pl.BlockSpec((D, 2 * D), lambda i: (0, 0)),
            pl.BlockSpec((1, D), lambda i: (0, 0)),
        ],
        out_specs=pl.BlockSpec((MB, D), lambda i: (i, 0)),
        out_shape=jax.ShapeDtypeStruct((B, D), f32),
    )(self_f, neigh_f, W1, b1.reshape(1, D))

    return out


# speedup vs baseline: 1.8139x; 1.2201x over previous
import functools

import jax
import jax.numpy as jnp
from jax import lax
from jax.experimental import pallas as pl
from jax.experimental.pallas import tpu as pltpu
from jax.experimental.pallas import tpu_sc as plsc

D = 128
DH = 64
L = 16
NC = 2
NS = 16
NW = NC * NS
N_PAD = 10240
ROWS_PER_TILE = N_PAD // NS
DEG_ROWS = N_PAD // L
K_E = 128
K_B = 80
B_PAD = 10240
CB = B_PAD // (NW * K_B)
MB = 1000

_mesh = plsc.VectorSubcoreMesh(core_axis_name="c", subcore_axis_name="s")
_sc_params = pltpu.CompilerParams(use_tc_tiling_on_sc=False,
                                  needs_layout_passes=False)


def _edge_kernel(src_hbm, dst_hbm, featS, zsum_hbm, zdeg_hbm, iota_hbm,
                 sum_out, deg_out,
                 src_v, dst_v, rows0, rows1,
                 degl_v, degf_v, iota_v,
                 acc_sp, deg_sp, g0, g1):
    c = lax.axis_index("c")
    s = lax.axis_index("s")
    rows = pl.ds(s * ROWS_PER_TILE, ROWS_PER_TILE)
    rows_out = pl.ds(c * N_PAD + s * ROWS_PER_TILE, ROWS_PER_TILE)
    drows = pl.ds(s * (DEG_ROWS // NS), DEG_ROWS // NS)
    pltpu.sync_copy(zsum_hbm.at[rows], acc_sp.at[rows])
    pltpu.sync_copy(zdeg_hbm.at[drows], deg_sp.at[drows])
    pltpu.sync_copy(zdeg_hbm, degl_v)
    pltpu.sync_copy(src_hbm.at[c, s], src_v)
    pltpu.sync_copy(dst_hbm.at[s], dst_v)
    pltpu.sync_copy(iota_hbm, iota_v)
    plsc.subcore_barrier()

    n_chunks = src_v.shape[0]
    bufs = [rows0, rows1]
    gsem = [g0, g1]
    ones16 = jnp.full((L,), 1.0, jnp.float32)

    def deg_chunk(j):
        def dseg(k, carry):
            dvec = dst_v[j, pl.ds(k * L, L)]
            plsc.addupdate_scatter(
                degl_v, [lax.shift_right_logical(dvec, 4), dvec & 15], ones16)
            return carry

        lax.fori_loop(0, K_E // L, dseg, 0)

    def start_g(j, b):
        pltpu.async_copy(featS.at[src_v.at[j]], bufs[b], gsem[b])

    def wait_g(b):
        pltpu.make_async_copy(featS.at[src_v.at[0]], bufs[b], gsem[b]).wait()

    start_g(0, 0)

    def pair(j2, carry):
        j0 = 2 * j2
        j1 = j0 + 1
        start_g(j1, 1)
        wait_g(0)
        pltpu.sync_copy(rows0, acc_sp.at[dst_v.at[j0]], add=True)
        deg_chunk(j0)
        start_g(jnp.minimum(j0 + 2, n_chunks - 1), 0)
        wait_g(1)
        pltpu.sync_copy(rows1, acc_sp.at[dst_v.at[j1]], add=True)
        deg_chunk(j1)
        return carry

    lax.fori_loop(0, n_chunks // 2, pair, 0)
    wait_g(0)

    def dmerge(r, carry):
        pltpu.sync_copy(degl_v.at[pl.ds(r * 128, 128)],
                        deg_sp.at[iota_v.at[r]], add=True)
        return carry

    lax.fori_loop(0, DEG_ROWS // 128, dmerge, 0)
    plsc.subcore_barrier()

    pltpu.sync_copy(acc_sp.at[rows], sum_out.at[rows_out])
    pltpu.sync_copy(deg_sp.at[drows], degf_v)

    def dexp(n, carry):
        val = plsc.load_gather(
            degf_v, [jnp.full((L,), n // L, jnp.int32),
                     jnp.full((L,), n % L, jnp.int32)])
        degl_v[n, :] = val
        return carry

    lax.fori_loop(0, ROWS_PER_TILE, dexp, 0)
    pltpu.sync_copy(degl_v.at[pl.ds(0, ROWS_PER_TILE)],
                    deg_out.at[pl.ds(s * ROWS_PER_TILE, ROWS_PER_TILE)])


def _combine_kernel(nodes_hbm, feat_hbm, sum_hbm, dg_hbm,
                    self_out, neigh_out,
                    nodes_v, idx2_v, sf_v, pa_v, pb_v, d_v, nout_v, sem):
    c = lax.axis_index("c")
    s = lax.axis_index("s")
    wid = s * NC + c
    pltpu.sync_copy(nodes_hbm.at[wid], nodes_v)
    base = wid * (CB * K_B)

    def chunk(j, carry):
        idx = nodes_v.at[j]

        def shift(kk, carry2):
            sl = pl.ds(kk * L, L)
            idx2_v[sl] = nodes_v[j, sl] + N_PAD
            return carry2

        lax.fori_loop(0, K_B // L, shift, 0)

        cps = [
            pltpu.async_copy(feat_hbm.at[idx], sf_v, sem),
            pltpu.async_copy(sum_hbm.at[idx], pa_v, sem),
            pltpu.async_copy(sum_hbm.at[idx2_v], pb_v, sem),
            pltpu.async_copy(dg_hbm.at[idx], d_v, sem),
        ]
        for cp in cps:
            cp.wait()

        def row(i, carry2):
            inv = 1.0 / jnp.maximum(d_v[i, :], 1.0)

            def seg(k, carry3):
                sl = pl.ds(k * L, L)
                nout_v[i, sl] = pa_v[i, sl] * inv
                nout_v[i, pl.ds(DH + k * L, L)] = pb_v[i, sl] * inv
                return carry3

            lax.fori_loop(0, DH // L, seg, 0)
            return carry2

        lax.fori_loop(0, K_B, row, 0)
        dst = pl.ds(base + j * K_B, K_B)
        pltpu.sync_copy(sf_v, self_out.at[dst])
        pltpu.sync_copy(nout_v, neigh_out.at[dst])
        return carry

    lax.fori_loop(0, CB, chunk, 0)


def _mlp_kernel(s_ref, n_ref, w_ref, b_ref, o_ref):
    xs = s_ref[...]
    xn = n_ref[...]
    w = w_ref[...]
    acc = lax.dot_general(xs, w[:, :D], (((1,), (1,)), ((), ())),
                          preferred_element_type=jnp.float32)
    acc += lax.dot_general(xn, w[:, D:], (((1,), (1,)), ((), ())),
                           preferred_element_type=jnp.float32)
    o_ref[...] = jnp.maximum(acc + b_ref[...], 0.0)


def _build_edge_kernel(ch2):
    f32 = jnp.float32
    i32 = jnp.int32
    return pl.kernel(
        _edge_kernel,
        out_type=[
            jax.ShapeDtypeStruct((2 * N_PAD, DH), f32),
            jax.ShapeDtypeStruct((N_PAD, 16), f32),
        ],
        mesh=_mesh,
        scratch_types=[
            pltpu.VMEM((ch2, K_E), i32),
            pltpu.VMEM((ch2, K_E), i32),
            pltpu.VMEM((K_E, DH), f32),
            pltpu.VMEM((K_E, DH), f32),
            pltpu.VMEM((DEG_ROWS, L), f32),
            pltpu.VMEM((DEG_ROWS // NS, L), f32),
            pltpu.VMEM((DEG_ROWS // 128, 128), i32),
            pltpu.VMEM_SHARED((N_PAD, DH), f32),
            pltpu.VMEM_SHARED((DEG_ROWS, L), f32),
            pltpu.SemaphoreType.DMA,
            pltpu.SemaphoreType.DMA,
        ],
        compiler_params=_sc_params,
    )


def _build_combine_kernel():
    f32 = jnp.float32
    return pl.kernel(
        _combine_kernel,
        out_type=[
            jax.ShapeDtypeStruct((B_PAD, D), f32),
            jax.ShapeDtypeStruct((B_PAD, D), f32),
        ],
        mesh=_mesh,
        scratch_types=[
            pltpu.VMEM((CB, K_B), jnp.int32),
            pltpu.VMEM((K_B,), jnp.int32),
            pltpu.VMEM((K_B, D), f32),
            pltpu.VMEM((K_B, DH), f32),
            pltpu.VMEM((K_B, DH), f32),
            pltpu.VMEM((K_B, 16), f32),
            pltpu.VMEM((K_B, D), f32),
            pltpu.SemaphoreType.DMA,
        ],
        compiler_params=_sc_params,
    )


def kernel(nodes, edge_index, features, W1, b1):
    E = edge_index.shape[1]
    B = nodes.shape[0]
    N = features.shape[0]
    f32 = jnp.float32
    i32 = jnp.int32

    e_per_tile = E // NS
    ch2 = -(-e_per_tile // (2 * K_E)) * 2
    e_pad = ch2 * K_E - e_per_tile

    src_t = jnp.pad(edge_index[0].astype(i32).reshape(NS, e_per_tile),
                    ((0, 0), (0, e_pad))).reshape(NS, ch2, K_E)
    pad_dst = (N + jnp.arange(e_pad, dtype=i32) % (N_PAD - N))[None, :]
    dst_t = jnp.concatenate(
        [edge_index[1].astype(i32).reshape(NS, e_per_tile),
         jnp.broadcast_to(pad_dst, (NS, e_pad))], axis=1).reshape(NS, ch2, K_E)
    src_both = jnp.stack([src_t, src_t + N])
    featS = jnp.concatenate([features[:, :DH], features[:, DH:]], axis=0)
    nodes_r = jnp.pad(nodes.astype(i32), (0, B_PAD - B)).reshape(NW, CB, K_B)
    zsum = jnp.zeros((N_PAD, DH), f32)
    zdeg = jnp.zeros((DEG_ROWS, L), f32)
    iota = jnp.arange(DEG_ROWS, dtype=i32).reshape(DEG_ROWS // 128, 128)

    sum_out, deg_out = _build_edge_kernel(ch2)(
        src_both, dst_t, featS, zsum, zdeg, iota)

    self_f, neigh_f = _build_combine_kernel()(
        nodes_r, features, sum_out, deg_out)

    out = pl.pallas_call(
        _mlp_kernel,
        grid=(B // MB,),
        in_specs=[
            pl.BlockSpec((MB, D), lambda i: (i, 0)),
            pl.BlockSpec((MB, D), lambda i: (i, 0)),
            pl.BlockSpec((D, 2 * D), lambda i: (0, 0)),
            pl.BlockSpec((1, D), lambda i: (0, 0)),
        ],
        out_specs=pl.BlockSpec((MB, D), lambda i: (i, 0)),
        out_shape=jax.ShapeDtypeStruct((B, D), f32),
    )(self_f, neigh_f, W1, b1.reshape(1, D))

    return out
